# baseline (speedup 1.0000x reference)
import functools

import jax
import jax.numpy as jnp
from jax import lax
from jax.experimental import pallas as pl
from jax.experimental.pallas import tpu as pltpu
from jax.experimental.pallas import tpu_sc as plsc

N_NODES = 10000
N_EDGES = 320000
D_FEAT = 128

NC = 2
NS = 16
NW = NC * NS

L = 128
E_RAW = N_EDGES // NW
CH_FULL = E_RAW // L
TAIL = E_RAW - CH_FULL * L
N_PAD = 10240
Z_W = N_PAD // NS

ROWS_BLK = 1024
N_BLKS = (N_NODES + ROWS_BLK - 1) // ROWS_BLK

_mesh = plsc.VectorSubcoreMesh(core_axis_name="c", subcore_axis_name="s")


@functools.partial(
    pl.kernel,
    out_type=jax.ShapeDtypeStruct((NC, N_PAD), jnp.float32),
    mesh=_mesh,
    scratch_types=[
        pltpu.VMEM((E_RAW,), jnp.int32),
        pltpu.VMEM((L,), jnp.float32),
        pltpu.VMEM((Z_W,), jnp.float32),
        pltpu.VMEM_SHARED((N_PAD,), jnp.float32),
    ],
)
def _degree_kernel(edge_hbm, out_hbm, idx_v, ones_v, zeros_v, hist_sh):
    cid = lax.axis_index("c")
    sid = lax.axis_index("s")
    wid = sid * NC + cid

    def fill(i, _):
        ones_v[pl.ds(i * 16, 16)] = jnp.ones((16,), jnp.float32)
        return 0

    lax.fori_loop(0, L // 16, fill, 0)

    def fillz(i, _):
        zeros_v[pl.ds(i * 16, 16)] = jnp.zeros((16,), jnp.float32)
        return 0

    lax.fori_loop(0, Z_W // 16, fillz, 0)

    pltpu.sync_copy(zeros_v, hist_sh.at[pl.ds(sid * Z_W, Z_W)])
    pltpu.sync_copy(edge_hbm.at[pl.ds(N_EDGES + wid * E_RAW, E_RAW)], idx_v)
    plsc.subcore_barrier()

    def scat(j, _):
        pltpu.sync_copy(ones_v, hist_sh.at[idx_v.at[pl.ds(j * L, L)]],
                        add=True)
        return 0

    lax.fori_loop(0, CH_FULL, scat, 0)
    pltpu.sync_copy(ones_v.at[pl.ds(0, TAIL)],
                    hist_sh.at[idx_v.at[pl.ds(CH_FULL * L, TAIL)]],
                    add=True)
    plsc.subcore_barrier()

    @pl.when(sid == 0)
    def _():
        pltpu.sync_copy(hist_sh, out_hbm.at[cid])


def _mm_body(x_ref, w_ref, o_ref):
    o_ref[...] = jnp.dot(x_ref[...], w_ref[...],
                         preferred_element_type=jnp.float32)


def _scale_body(c_ref, xw_ref, o_ref):
    cnt = c_ref[0, :] + c_ref[1, :]
    o_ref[...] = xw_ref[...] * cnt[:, None]


def kernel(edge_index, x, W):
    edges_flat = edge_index.reshape(2 * N_EDGES)
    deg = _degree_kernel(edges_flat)

    xw = pl.pallas_call(
        _mm_body,
        grid=(N_BLKS,),
        out_shape=jax.ShapeDtypeStruct((N_NODES, D_FEAT), jnp.float32),
        in_specs=[
            pl.BlockSpec((ROWS_BLK, D_FEAT), lambda i: (i, 0)),
            pl.BlockSpec((D_FEAT, D_FEAT), lambda i: (0, 0)),
        ],
        out_specs=pl.BlockSpec((ROWS_BLK, D_FEAT), lambda i: (i, 0)),
    )(x, W)

    out = pl.pallas_call(
        _scale_body,
        grid=(N_BLKS,),
        out_shape=jax.ShapeDtypeStruct((N_NODES, D_FEAT), jnp.float32),
        in_specs=[
            pl.BlockSpec((NC, ROWS_BLK), lambda i: (0, i)),
            pl.BlockSpec((ROWS_BLK, D_FEAT), lambda i: (i, 0)),
        ],
        out_specs=pl.BlockSpec((ROWS_BLK, D_FEAT), lambda i: (i, 0)),
    )(deg, xw)
    return out


# speedup vs baseline: 1.0618x; 1.0618x over previous
import functools

import jax
import jax.numpy as jnp
from jax import lax
from jax.experimental import pallas as pl
from jax.experimental.pallas import tpu as pltpu
from jax.experimental.pallas import tpu_sc as plsc

N_NODES = 10000
N_EDGES = 320000
D_FEAT = 128

NC = 2
NS = 16
NW = NC * NS

E_RAW = N_EDGES // NW
N_PAD = 10240
Z_W = N_PAD // NS

ROWS_BLK = 1024
N_BLKS = (N_NODES + ROWS_BLK - 1) // ROWS_BLK

_mesh = plsc.VectorSubcoreMesh(core_axis_name="c", subcore_axis_name="s")


@functools.partial(
    pl.kernel,
    out_type=jax.ShapeDtypeStruct((NC, N_PAD), jnp.float32),
    mesh=_mesh,
    scratch_types=[
        pltpu.VMEM((E_RAW,), jnp.int32),
        pltpu.VMEM((E_RAW,), jnp.float32),
        pltpu.VMEM((Z_W,), jnp.float32),
        pltpu.VMEM_SHARED((N_PAD,), jnp.float32),
    ],
)
def _degree_kernel(edge_hbm, out_hbm, idx_v, ones_v, zeros_v, hist_sh):
    cid = lax.axis_index("c")
    sid = lax.axis_index("s")
    wid = sid * NC + cid

    def fill(i, _):
        ones_v[pl.ds(i * 16, 16)] = jnp.ones((16,), jnp.float32)
        return 0

    lax.fori_loop(0, E_RAW // 16, fill, 0)

    def fillz(i, _):
        zeros_v[pl.ds(i * 16, 16)] = jnp.zeros((16,), jnp.float32)
        return 0

    lax.fori_loop(0, Z_W // 16, fillz, 0)

    pltpu.sync_copy(zeros_v, hist_sh.at[pl.ds(sid * Z_W, Z_W)])
    pltpu.sync_copy(edge_hbm.at[pl.ds(N_EDGES + wid * E_RAW, E_RAW)], idx_v)
    plsc.subcore_barrier()

    pltpu.sync_copy(ones_v, hist_sh.at[idx_v], add=True)
    plsc.subcore_barrier()

    @pl.when(sid == 0)
    def _():
        pltpu.sync_copy(hist_sh, out_hbm.at[cid])


def _mm_body(x_ref, w_ref, o_ref):
    o_ref[...] = jnp.dot(x_ref[...], w_ref[...],
                         preferred_element_type=jnp.float32)


def _scale_body(c_ref, xw_ref, o_ref):
    cnt = c_ref[0, :] + c_ref[1, :]
    o_ref[...] = xw_ref[...] * cnt[:, None]


def kernel(edge_index, x, W):
    edges_flat = edge_index.reshape(2 * N_EDGES)
    deg = _degree_kernel(edges_flat)

    xw = pl.pallas_call(
        _mm_body,
        grid=(N_BLKS,),
        out_shape=jax.ShapeDtypeStruct((N_NODES, D_FEAT), jnp.float32),
        in_specs=[
            pl.BlockSpec((ROWS_BLK, D_FEAT), lambda i: (i, 0)),
            pl.BlockSpec((D_FEAT, D_FEAT), lambda i: (0, 0)),
        ],
        out_specs=pl.BlockSpec((ROWS_BLK, D_FEAT), lambda i: (i, 0)),
    )(x, W)

    out = pl.pallas_call(
        _scale_body,
        grid=(N_BLKS,),
        out_shape=jax.ShapeDtypeStruct((N_NODES, D_FEAT), jnp.float32),
        in_specs=[
            pl.BlockSpec((NC, ROWS_BLK), lambda i: (0, i)),
            pl.BlockSpec((ROWS_BLK, D_FEAT), lambda i: (i, 0)),
        ],
        out_specs=pl.BlockSpec((ROWS_BLK, D_FEAT), lambda i: (i, 0)),
    )(deg, xw)
    return out


# speedup vs baseline: 1.0741x; 1.0117x over previous
import functools

import jax
import jax.numpy as jnp
from jax import lax
from jax.experimental import pallas as pl
from jax.experimental.pallas import tpu as pltpu
from jax.experimental.pallas import tpu_sc as plsc

N_NODES = 10000
N_EDGES = 320000
D_FEAT = 128

NC = 2
NS = 16
NW = NC * NS

E_RAW = N_EDGES // NW
N_PAD = 10240
Z_W = N_PAD // NS

ROWS_BLK = 1024
N_BLKS = (N_NODES + ROWS_BLK - 1) // ROWS_BLK

_mesh = plsc.VectorSubcoreMesh(core_axis_name="c", subcore_axis_name="s")


@functools.partial(
    pl.kernel,
    out_type=jax.ShapeDtypeStruct((NC, N_PAD), jnp.float32),
    mesh=_mesh,
    scratch_types=[
        pltpu.VMEM((E_RAW,), jnp.int32),
        pltpu.VMEM((E_RAW,), jnp.float32),
        pltpu.VMEM((Z_W,), jnp.float32),
        pltpu.VMEM_SHARED((N_PAD,), jnp.float32),
    ],
)
def _degree_kernel(edge_hbm, out_hbm, idx_v, ones_v, zeros_v, hist_sh):
    cid = lax.axis_index("c")
    sid = lax.axis_index("s")
    wid = sid * NC + cid

    def fill(i, _):
        ones_v[pl.ds(i * 16, 16)] = jnp.ones((16,), jnp.float32)
        return 0

    lax.fori_loop(0, E_RAW // 16, fill, 0)

    def fillz(i, _):
        zeros_v[pl.ds(i * 16, 16)] = jnp.zeros((16,), jnp.float32)
        return 0

    lax.fori_loop(0, Z_W // 16, fillz, 0)

    pltpu.sync_copy(zeros_v, hist_sh.at[pl.ds(sid * Z_W, Z_W)])
    pltpu.sync_copy(edge_hbm.at[pl.ds(N_EDGES + wid * E_RAW, E_RAW)], idx_v)
    plsc.subcore_barrier()

    pltpu.sync_copy(ones_v, hist_sh.at[idx_v], add=True)
    plsc.subcore_barrier()

    @pl.when(sid == 0)
    def _():
        pltpu.sync_copy(hist_sh, out_hbm.at[cid])


def _mm_scale_body(c_ref, x_ref, w_ref, o_ref):
    cnt = c_ref[0, :] + c_ref[1, :]
    xw = jnp.dot(x_ref[...], w_ref[...],
                 preferred_element_type=jnp.float32)
    o_ref[...] = xw * cnt[:, None]


def kernel(edge_index, x, W):
    edges_flat = edge_index.reshape(2 * N_EDGES)
    deg = _degree_kernel(edges_flat)

    out = pl.pallas_call(
        _mm_scale_body,
        grid=(N_BLKS,),
        out_shape=jax.ShapeDtypeStruct((N_NODES, D_FEAT), jnp.float32),
        in_specs=[
            pl.BlockSpec((NC, ROWS_BLK), lambda i: (0, i)),
            pl.BlockSpec((ROWS_BLK, D_FEAT), lambda i: (i, 0)),
            pl.BlockSpec((D_FEAT, D_FEAT), lambda i: (0, 0)),
        ],
        out_specs=pl.BlockSpec((ROWS_BLK, D_FEAT), lambda i: (i, 0)),
    )(deg, x, W)
    return out
